# baseline (speedup 1.0000x reference)
import math

import jax
import jax.numpy as jnp
from jax.experimental import pallas as pl

N = 10000
E = 320000
F = 128
D = 256
G = 512
CLS = 16


def _bn(x, g, b):
    mean = jnp.mean(x, axis=0)
    var = jnp.var(x, axis=0)
    return (x - mean) / jnp.sqrt(var + 1e-5) * g + b


def _tconv(x, src, dst, Wq, bq, Wk, bk, Wv, bv, Ws, bs):
    H, C = 1, D
    q = (x @ Wq + bq).reshape(N, H, C)
    k = (x @ Wk + bk).reshape(N, H, C)
    v = (x @ Wv + bv).reshape(N, H, C)
    logits = jnp.sum(q[dst] * k[src], axis=-1) / math.sqrt(C)
    m = jax.ops.segment_max(logits, dst, num_segments=N)
    m = jnp.where(jnp.isfinite(m), m, 0.0)
    ex = jnp.exp(logits - m[dst])
    den = jax.ops.segment_sum(ex, dst, num_segments=N)
    alpha = ex / (den[dst] + 1e-16)
    out = jax.ops.segment_sum(alpha[:, :, None] * v[src], dst, num_segments=N)
    return out.reshape(N, H * C) + x @ Ws + bs


def _matmul_kernel(x_ref, w_ref, o_ref):
    o_ref[...] = jnp.dot(x_ref[...], w_ref[...],
                         preferred_element_type=jnp.float32)


def _pallas_matmul(x, w):
    return pl.pallas_call(
        _matmul_kernel,
        out_shape=jax.ShapeDtypeStruct((x.shape[0], w.shape[1]), jnp.float32),
    )(x, w)


def kernel(x, params, edge_index, batch):
    p = params
    src, dst = edge_index[0], edge_index[1]
    x = x @ p["W_lin"] + p["b_lin"]
    x = jax.nn.relu(_bn(x, p["bn0_g"], p["bn0_b"]))
    for i in range(5):
        x = _tconv(x, src, dst,
                   p["c%d_Wq" % i], p["c%d_bq" % i],
                   p["c%d_Wk" % i], p["c%d_bk" % i],
                   p["c%d_Wv" % i], p["c%d_bv" % i],
                   p["c%d_Ws" % i], p["c%d_bs" % i])
        x = jax.nn.relu(_bn(x, p["bn%d_g" % (i + 1)], p["bn%d_b" % (i + 1)]))
    s = jax.ops.segment_sum(x, batch, num_segments=G)
    cnt = jax.ops.segment_sum(jnp.ones((N,), jnp.float32), batch, num_segments=G)
    pooled = s / jnp.maximum(cnt, 1.0)[:, None]
    return _pallas_matmul(pooled, p["W_out"]) + p["b_out"]


# speedup vs baseline: 2.7169x; 2.7169x over previous
import functools
import math

import jax
import jax.numpy as jnp
from jax import lax
from jax.experimental import pallas as pl
from jax.experimental.pallas import tpu as pltpu
from jax.experimental.pallas import tpu_sc as plsc

N = 10000
E = 320000
F = 128
D = 256
G = 512
CLS = 16

NW = 32
NPW = 320
NPAD = NW * NPW
EPAD = E + 64
BR = 1024
NRB = NPAD // BR
CHK = 32
QB = 64
LANES = 16
DB = D // LANES

import numpy as _np
_GDN = lax.GatherDimensionNumbers(offset_dims=(), collapsed_slice_dims=(0,),
                                  start_index_map=(0,))


def _lane_bfly_sum(x):
    iota = lax.iota(jnp.int32, LANES)
    for s in (1, 2, 4, 8):
        idx = jnp.bitwise_xor(iota, s).reshape(LANES, 1)
        x = x + lax.gather(x, idx, _GDN, slice_sizes=(1,),
                           mode=lax.GatherScatterMode.PROMISE_IN_BOUNDS)
    return x


def _qkvs_body(x_ref, w_ref, b_ref, oq, ok, ov, os_, qn2, kn2):
    y = jnp.dot(x_ref[...], w_ref[...], preferred_element_type=jnp.float32)
    y = y + b_ref[...]
    q = y[:, 0 * D:1 * D]
    k = y[:, 1 * D:2 * D]
    oq[...] = q
    ok[...] = k
    ov[...] = y[:, 2 * D:3 * D]
    os_[...] = y[:, 3 * D:4 * D]
    qn2[...] = jnp.sum(q * q, axis=1).reshape(BR // 128, 128)
    kn2[...] = jnp.sum(k * k, axis=1).reshape(BR // 128, 128)


def _qkvs_matmul(x, wcat, bcat):
    din = x.shape[1]
    grid = (NRB,)
    out_shapes = [jax.ShapeDtypeStruct((NPAD, D), jnp.float32)] * 4 + [
        jax.ShapeDtypeStruct((NPAD // 128, 128), jnp.float32)] * 2
    out_specs = [pl.BlockSpec((BR, D), lambda i: (i, 0))] * 4 + [
        pl.BlockSpec((BR // 128, 128), lambda i: (i, 0))] * 2
    return pl.pallas_call(
        _qkvs_body,
        grid=grid,
        in_specs=[
            pl.BlockSpec((BR, din), lambda i: (i, 0)),
            pl.BlockSpec((din, 4 * D), lambda i: (0, 0)),
            pl.BlockSpec((1, 4 * D), lambda i: (0, 0)),
        ],
        out_specs=out_specs,
        out_shape=out_shapes,
    )(x, wcat, bcat)


def _bn_body(a_ref, r_ref, g_ref, b_ref, o_ref, s_ref, ss_ref, *, ncols):
    p = pl.program_id(0)
    j = pl.program_id(1)

    h = a_ref[...] + r_ref[...]
    rows = j * BR + lax.broadcasted_iota(jnp.int32, (BR, 1), 0)
    mask = (rows < N).astype(jnp.float32)

    @pl.when((p == 0) & (j == 0))
    def _():
        s_ref[...] = jnp.zeros_like(s_ref)
        ss_ref[...] = jnp.zeros_like(ss_ref)

    @pl.when(p == 0)
    def _():
        hm = h * mask
        s_ref[...] += jnp.sum(hm, axis=0, keepdims=True)
        ss_ref[...] += jnp.sum(hm * h, axis=0, keepdims=True)

    @pl.when(p == 1)
    def _():
        mean = s_ref[...] / N
        var = ss_ref[...] / N - mean * mean
        inv = lax.rsqrt(var + 1e-5)
        o_ref[...] = jnp.maximum((h - mean) * inv * g_ref[...] + b_ref[...],
                                 0.0)


def _bn_relu(a, r, g, b):
    ncols = a.shape[1]
    return pl.pallas_call(
        functools.partial(_bn_body, ncols=ncols),
        grid=(2, NRB),
        in_specs=[
            pl.BlockSpec((BR, ncols), lambda p, j: (j, 0)),
            pl.BlockSpec((BR, ncols), lambda p, j: (j, 0)),
            pl.BlockSpec((1, ncols), lambda p, j: (0, 0)),
            pl.BlockSpec((1, ncols), lambda p, j: (0, 0)),
        ],
        out_specs=pl.BlockSpec((BR, ncols), lambda p, j: (j, 0)),
        out_shape=jax.ShapeDtypeStruct((NPAD, ncols), jnp.float32),
        scratch_shapes=[
            pltpu.VMEM((1, ncols), jnp.float32),
            pltpu.VMEM((1, ncols), jnp.float32),
        ],
    )(a, r, g, b)


def _pool_body(x_ref, b_ref, s_ref, c_ref):
    j = pl.program_id(0)
    onehot = (b_ref[...] == lax.broadcasted_iota(jnp.int32, (BR, G), 1)
              ).astype(jnp.float32)
    part = lax.dot_general(onehot, x_ref[...], (((0,), (0,)), ((), ())),
                           preferred_element_type=jnp.float32)
    cnt = lax.dot_general(onehot, jnp.ones((BR, 128), jnp.float32),
                          (((0,), (0,)), ((), ())),
                          preferred_element_type=jnp.float32)

    @pl.when(j == 0)
    def _():
        s_ref[...] = jnp.zeros_like(s_ref)
        c_ref[...] = jnp.zeros_like(c_ref)

    s_ref[...] += part
    c_ref[...] += cnt


def _pool(x, batch_pad):
    return pl.pallas_call(
        _pool_body,
        grid=(NRB,),
        in_specs=[
            pl.BlockSpec((BR, D), lambda j: (j, 0)),
            pl.BlockSpec((BR, 1), lambda j: (j, 0)),
        ],
        out_specs=[
            pl.BlockSpec((G, D), lambda j: (0, 0)),
            pl.BlockSpec((G, 128), lambda j: (0, 0)),
        ],
        out_shape=[
            jax.ShapeDtypeStruct((G, D), jnp.float32),
            jax.ShapeDtypeStruct((G, 128), jnp.float32),
        ],
    )(x, batch_pad)


def _head_body(s_ref, c_ref, w_ref, b_ref, o_ref):
    cnt = c_ref[:, 0:1]
    pooled = s_ref[...] / jnp.maximum(cnt, 1.0)
    o_ref[...] = jnp.dot(pooled, w_ref[...],
                         preferred_element_type=jnp.float32) + b_ref[...]


def _head(sums, cnts, w, b):
    return pl.pallas_call(
        _head_body,
        out_shape=jax.ShapeDtypeStruct((G, CLS), jnp.float32),
    )(sums, cnts, w, b)


def _edge_body(q_hbm, k_hbm, v_hbm, src_hbm, dst_hbm, rs_hbm, bnd_hbm,
               out_hbm, qblk, obuf, kbuf, vbuf, srcb, dstb, rsb, bb,
               semk, semv):
    wid = lax.axis_index("s") * 2 + lax.axis_index("c")
    n0 = pl.multiple_of(wid * NPW, NPW)

    pltpu.sync_copy(rs_hbm.at[pl.ds(n0, NPW + 8)], rsb)
    pltpu.sync_copy(
        bnd_hbm.at[pl.ds(pl.multiple_of(n0 * LANES, NPW * LANES),
                         NPW * LANES)], bb)

    def _zb(i, _):
        obuf[pl.ds(i * LANES, LANES)] = jnp.zeros((LANES,), jnp.float32)
        return 0
    lax.fori_loop(0, NPW * DB, _zb, 0)

    e0 = rsb[pl.ds(0, LANES)][0]
    e1 = rsb[pl.ds(NPW - 8, LANES)][8]
    a0 = lax.bitwise_and(e0, jnp.int32(-8))
    nch = lax.div(e1 - a0 + (CHK - 1), jnp.int32(CHK))

    zero16 = jnp.zeros((LANES,), jnp.float32)

    def chunk_body(c, st):
        cur, row, qb, den = st[0], st[1], st[2], st[3]
        acc = list(st[4])
        cs = pl.multiple_of(a0 + c * CHK, 8)
        pltpu.sync_copy(src_hbm.at[pl.ds(cs, CHK)], srcb)
        pltpu.sync_copy(dst_hbm.at[pl.ds(cs, CHK)], dstb)
        cpk = pltpu.async_copy(k_hbm.at[srcb], kbuf, semk)
        cpv = pltpu.async_copy(v_hbm.at[srcb], vbuf, semv)
        cpk.wait()
        cpv.wait()

        dvecs = [dstb[pl.ds(u * LANES, LANES)] for u in range(CHK // LANES)]

        for j in range(CHK):
            eabs = cs + j
            valid = (eabs >= e0) & (eabs < e1)
            d = dvecs[j // LANES][j % LANES]
            sw = valid & (d != cur)
            nqb = lax.div(d - n0, jnp.int32(QB))

            @pl.when(sw)
            def _(cur=cur, row=row, qb=qb, den=den, acc=acc, nqb=nqb):
                @pl.when(cur >= 0)
                def _():
                    r = 1.0 / (den + 1e-16)
                    for i in range(DB):
                        obuf[pl.ds(row * D + i * LANES, LANES)] = acc[i] * r

                @pl.when(nqb != qb)
                def _():
                    qstart = pl.multiple_of((n0 + nqb * QB) * D, QB * D)
                    pltpu.sync_copy(q_hbm.at[pl.ds(qstart, QB * D)], qblk)

            cur = jnp.where(sw, d, cur)
            row = jnp.where(sw, d - n0, row)
            qb = jnp.where(sw, nqb, qb)
            den = jnp.where(sw, zero16, den)
            acc = [jnp.where(sw, zero16, a) for a in acc]

            qoff = jnp.clip((row - qb * QB) * D, 0, (QB - 1) * D)
            part = (qblk[pl.ds(qoff, LANES)] * kbuf[j, pl.ds(0, LANES)])
            for i in range(1, DB):
                part = part + (qblk[pl.ds(qoff + i * LANES, LANES)]
                               * kbuf[j, pl.ds(i * LANES, LANES)])
            l = _lane_bfly_sum(part) * (1.0 / math.sqrt(D))
            bsp = bb[pl.ds(row * LANES, LANES)]
            ex = jnp.exp(l - bsp)
            ex = jnp.where(valid, ex, zero16)
            den = den + ex
            for i in range(DB):
                acc[i] = acc[i] + ex * vbuf[j, pl.ds(i * LANES, LANES)]

        return (cur, row, qb, den, tuple(acc))

    init = (jnp.int32(-1), jnp.int32(0), jnp.int32(-1), zero16,
            tuple([zero16] * DB))
    cur, row, qb, den, acc = lax.fori_loop(0, nch, chunk_body, init)

    @pl.when(cur >= 0)
    def _():
        r = 1.0 / (den + 1e-16)
        for i in range(DB):
            obuf[pl.ds(row * D + i * LANES, LANES)] = acc[i] * r

    pltpu.sync_copy(obuf, out_hbm.at[pl.ds(pl.multiple_of(n0 * D, NPW * D),
                                           NPW * D)])


def _edge_stage(q1d, k2d, v2d, srcp, dstp, rs, bnd):
    mesh = plsc.VectorSubcoreMesh(core_axis_name="c", subcore_axis_name="s")
    f = pl.kernel(
        _edge_body,
        out_type=jax.ShapeDtypeStruct((NPAD * D,), jnp.float32),
        mesh=mesh,
        scratch_types=[
            pltpu.VMEM((QB * D,), jnp.float32),
            pltpu.VMEM((NPW * D,), jnp.float32),
            pltpu.VMEM((CHK, D), jnp.float32),
            pltpu.VMEM((CHK, D), jnp.float32),
            pltpu.VMEM((CHK,), jnp.int32),
            pltpu.VMEM((CHK,), jnp.int32),
            pltpu.VMEM((NPW + 8,), jnp.int32),
            pltpu.VMEM((NPW * LANES,), jnp.float32),
            pltpu.SemaphoreType.DMA,
            pltpu.SemaphoreType.DMA,
        ],
    )
    return f(q1d, k2d, v2d, srcp, dstp, rs, bnd)


def kernel(x, params, edge_index, batch):
    p = params
    src, dst = edge_index[0], edge_index[1]

    perm = jnp.argsort(dst)
    srcp = jnp.pad(src[perm], (0, EPAD - E))
    dstp_real = dst[perm]
    dstp = jnp.pad(dstp_real, (0, EPAD - E))
    rs = jnp.searchsorted(dstp_real, jnp.arange(NPAD + 8, dtype=jnp.int32),
                          side="left").astype(jnp.int32)
    batch_pad = jnp.pad(batch, (0, NPAD - N),
                        constant_values=G).reshape(NPAD, 1)

    xp = jnp.pad(x, ((0, NPAD - N), (0, 0)))

    wlin = jnp.concatenate(
        [p["W_lin"]] + [jnp.zeros((F, F), jnp.float32)] * 3, axis=1)
    blin = jnp.concatenate(
        [p["b_lin"]] + [jnp.zeros((F,), jnp.float32)] * 3).reshape(1, 4 * F)
    h0 = _qkvs_matmul(xp, wlin, blin)[0][:, :F]
    zeros_f = jnp.zeros((NPAD, F), jnp.float32)
    h = _bn_relu(h0, zeros_f, p["bn0_g"].reshape(1, F),
                 p["bn0_b"].reshape(1, F))

    for i in range(5):
        wcat = jnp.concatenate(
            [p["c%d_W%s" % (i, nm)] for nm in ("q", "k", "v", "s")], axis=1)
        bcat = jnp.concatenate(
            [p["c%d_b%s" % (i, nm)] for nm in ("q", "k", "v", "s")]
        ).reshape(1, 4 * D)
        q, k, v, s, qn2, kn2 = _qkvs_matmul(h, wcat, bcat)
        kmax2 = jnp.max(kn2)
        bnd = (jnp.sqrt(qn2.reshape(NPAD)) * jnp.sqrt(kmax2)
               * (1.0 / math.sqrt(D)))
        bnd = jnp.broadcast_to(bnd[:, None], (NPAD, LANES)).reshape(-1)
        attn = _edge_stage(q.reshape(NPAD * D), k, v, srcp, dstp, rs, bnd)
        h = _bn_relu(attn.reshape(NPAD, D), s,
                     p["bn%d_g" % (i + 1)].reshape(1, D),
                     p["bn%d_b" % (i + 1)].reshape(1, D))

    sums, cnts = _pool(h, batch_pad)
    wout = p["W_out"]
    bout = p["b_out"].reshape(1, CLS)
    return _head(sums, cnts, wout, bout)


# speedup vs baseline: 2.9788x; 1.0964x over previous
import functools
import math

import jax
import jax.numpy as jnp
from jax import lax
from jax.experimental import pallas as pl
from jax.experimental.pallas import tpu as pltpu
from jax.experimental.pallas import tpu_sc as plsc

N = 10000
E = 320000
F = 128
D = 256
G = 512
CLS = 16

NW = 32
NPW = 320
NPAD = NW * NPW
EPAD = E + 192
BR = 1024
NRB = NPAD // BR
CHK = 32
QB = 32
LANES = 16
DB = D // LANES

import numpy as _np
_GDN = lax.GatherDimensionNumbers(offset_dims=(), collapsed_slice_dims=(0,),
                                  start_index_map=(0,))


def _lane_bfly_sum(x):
    iota = lax.iota(jnp.int32, LANES)
    for s in (1, 2, 4, 8):
        idx = jnp.bitwise_xor(iota, s).reshape(LANES, 1)
        x = x + lax.gather(x, idx, _GDN, slice_sizes=(1,),
                           mode=lax.GatherScatterMode.PROMISE_IN_BOUNDS)
    return x


def _qkvs_body(x_ref, w_ref, b_ref, oq, ok, ov, os_, qn2, kn2):
    y = jnp.dot(x_ref[...], w_ref[...], preferred_element_type=jnp.float32)
    y = y + b_ref[...]
    q = y[:, 0 * D:1 * D]
    k = y[:, 1 * D:2 * D]
    oq[...] = q
    ok[...] = k
    ov[...] = y[:, 2 * D:3 * D]
    os_[...] = y[:, 3 * D:4 * D]
    qn2[...] = jnp.sum(q * q, axis=1).reshape(BR // 128, 128)
    kn2[...] = jnp.sum(k * k, axis=1).reshape(BR // 128, 128)


def _qkvs_matmul(x, wcat, bcat):
    din = x.shape[1]
    grid = (NRB,)
    out_shapes = [jax.ShapeDtypeStruct((NPAD, D), jnp.float32)] * 4 + [
        jax.ShapeDtypeStruct((NPAD // 128, 128), jnp.float32)] * 2
    out_specs = [pl.BlockSpec((BR, D), lambda i: (i, 0))] * 4 + [
        pl.BlockSpec((BR // 128, 128), lambda i: (i, 0))] * 2
    return pl.pallas_call(
        _qkvs_body,
        grid=grid,
        in_specs=[
            pl.BlockSpec((BR, din), lambda i: (i, 0)),
            pl.BlockSpec((din, 4 * D), lambda i: (0, 0)),
            pl.BlockSpec((1, 4 * D), lambda i: (0, 0)),
        ],
        out_specs=out_specs,
        out_shape=out_shapes,
    )(x, wcat, bcat)


def _bn_body(a_ref, r_ref, g_ref, b_ref, o_ref, s_ref, ss_ref, *, ncols):
    p = pl.program_id(0)
    j = pl.program_id(1)

    h = a_ref[...] + r_ref[...]
    rows = j * BR + lax.broadcasted_iota(jnp.int32, (BR, 1), 0)
    mask = (rows < N).astype(jnp.float32)

    @pl.when((p == 0) & (j == 0))
    def _():
        s_ref[...] = jnp.zeros_like(s_ref)
        ss_ref[...] = jnp.zeros_like(ss_ref)

    @pl.when(p == 0)
    def _():
        hm = h * mask
        s_ref[...] += jnp.sum(hm, axis=0, keepdims=True)
        ss_ref[...] += jnp.sum(hm * h, axis=0, keepdims=True)

    @pl.when(p == 1)
    def _():
        mean = s_ref[...] / N
        var = ss_ref[...] / N - mean * mean
        inv = lax.rsqrt(var + 1e-5)
        o_ref[...] = jnp.maximum((h - mean) * inv * g_ref[...] + b_ref[...],
                                 0.0)


def _bn_relu(a, r, g, b):
    ncols = a.shape[1]
    return pl.pallas_call(
        functools.partial(_bn_body, ncols=ncols),
        grid=(2, NRB),
        in_specs=[
            pl.BlockSpec((BR, ncols), lambda p, j: (j, 0)),
            pl.BlockSpec((BR, ncols), lambda p, j: (j, 0)),
            pl.BlockSpec((1, ncols), lambda p, j: (0, 0)),
            pl.BlockSpec((1, ncols), lambda p, j: (0, 0)),
        ],
        out_specs=pl.BlockSpec((BR, ncols), lambda p, j: (j, 0)),
        out_shape=jax.ShapeDtypeStruct((NPAD, ncols), jnp.float32),
        scratch_shapes=[
            pltpu.VMEM((1, ncols), jnp.float32),
            pltpu.VMEM((1, ncols), jnp.float32),
        ],
    )(a, r, g, b)


def _pool_body(x_ref, b_ref, s_ref, c_ref):
    j = pl.program_id(0)
    onehot = (b_ref[...] == lax.broadcasted_iota(jnp.int32, (BR, G), 1)
              ).astype(jnp.float32)
    part = lax.dot_general(onehot, x_ref[...], (((0,), (0,)), ((), ())),
                           preferred_element_type=jnp.float32)
    cnt = lax.dot_general(onehot, jnp.ones((BR, 128), jnp.float32),
                          (((0,), (0,)), ((), ())),
                          preferred_element_type=jnp.float32)

    @pl.when(j == 0)
    def _():
        s_ref[...] = jnp.zeros_like(s_ref)
        c_ref[...] = jnp.zeros_like(c_ref)

    s_ref[...] += part
    c_ref[...] += cnt


def _pool(x, batch_pad):
    return pl.pallas_call(
        _pool_body,
        grid=(NRB,),
        in_specs=[
            pl.BlockSpec((BR, D), lambda j: (j, 0)),
            pl.BlockSpec((BR, 1), lambda j: (j, 0)),
        ],
        out_specs=[
            pl.BlockSpec((G, D), lambda j: (0, 0)),
            pl.BlockSpec((G, 128), lambda j: (0, 0)),
        ],
        out_shape=[
            jax.ShapeDtypeStruct((G, D), jnp.float32),
            jax.ShapeDtypeStruct((G, 128), jnp.float32),
        ],
    )(x, batch_pad)


def _head_body(s_ref, c_ref, w_ref, b_ref, o_ref):
    cnt = c_ref[:, 0:1]
    pooled = s_ref[...] / jnp.maximum(cnt, 1.0)
    o_ref[...] = jnp.dot(pooled, w_ref[...],
                         preferred_element_type=jnp.float32) + b_ref[...]


def _head(sums, cnts, w, b):
    return pl.pallas_call(
        _head_body,
        out_shape=jax.ShapeDtypeStruct((G, CLS), jnp.float32),
    )(sums, cnts, w, b)


def _edge_body(q_hbm, k_hbm, v_hbm, src_hbm, dst_hbm, rs_hbm, bnd_hbm,
               out_hbm, qblk, obuf, kbuf0, vbuf0, srcb0, dstb0, kbuf1,
               vbuf1, srcb1, dstb1, rsb, bb, semk0, semv0, semk1, semv1,
               semis0, semid0, semis1, semid1):
    wid = lax.axis_index("s") * 2 + lax.axis_index("c")
    n0 = pl.multiple_of(wid * NPW, NPW)

    kbuf = [kbuf0, kbuf1]
    vbuf = [vbuf0, vbuf1]
    srcb = [srcb0, srcb1]
    dstb = [dstb0, dstb1]
    semk = [semk0, semk1]
    semv = [semv0, semv1]
    semis = [semis0, semis1]
    semid = [semid0, semid1]

    pltpu.sync_copy(rs_hbm.at[pl.ds(n0, NPW + 8)], rsb)
    pltpu.sync_copy(
        bnd_hbm.at[pl.ds(pl.multiple_of(n0 * LANES, NPW * LANES),
                         NPW * LANES)], bb)

    def _zb(i, _):
        obuf[pl.ds(i * LANES, LANES)] = jnp.zeros((LANES,), jnp.float32)
        return 0
    lax.fori_loop(0, NPW * DB, _zb, 0)

    e0 = rsb[pl.ds(0, LANES)][0]
    e1 = rsb[pl.ds(NPW - 8, LANES)][8]
    a0 = lax.bitwise_and(e0, jnp.int32(-8))
    nch = lax.div(e1 - a0 + (CHK - 1), jnp.int32(CHK))

    zero16 = jnp.zeros((LANES,), jnp.float32)

    def _ids_start(c, b):
        cs = pl.multiple_of(a0 + c * CHK, 8)
        pltpu.async_copy(src_hbm.at[pl.ds(cs, CHK)], srcb[b], semis[b])
        pltpu.async_copy(dst_hbm.at[pl.ds(cs, CHK)], dstb[b], semid[b])

    def _ids_wait(b):
        pltpu.make_async_copy(src_hbm.at[pl.ds(0, CHK)], srcb[b],
                              semis[b]).wait()
        pltpu.make_async_copy(dst_hbm.at[pl.ds(0, CHK)], dstb[b],
                              semid[b]).wait()

    def _gather_start(b):
        pltpu.async_copy(k_hbm.at[srcb[b]], kbuf[b], semk[b])
        pltpu.async_copy(v_hbm.at[srcb[b]], vbuf[b], semv[b])

    def _gather_wait(b):
        pltpu.make_async_copy(k_hbm.at[srcb[b]], kbuf[b], semk[b]).wait()
        pltpu.make_async_copy(v_hbm.at[srcb[b]], vbuf[b], semv[b]).wait()

    @pl.when(nch > 0)
    def _():
        _ids_start(0, 0)
        _ids_wait(0)
        _gather_start(0)

    @pl.when(nch > 1)
    def _():
        _ids_start(1, 1)

    def compute_chunk(c, b, st):
        cur, row, qb, den = st[0], st[1], st[2], st[3]
        acc = list(st[4])
        cs = pl.multiple_of(a0 + c * CHK, 8)

        dvecs = [dstb[b][pl.ds(u * LANES, LANES)]
                 for u in range(CHK // LANES)]

        for j in range(CHK):
            eabs = cs + j
            valid = (eabs >= e0) & (eabs < e1)
            d = dvecs[j // LANES][j % LANES]
            sw = valid & (d != cur)
            nqb = lax.div(d - n0, jnp.int32(QB))

            @pl.when(sw)
            def _(cur=cur, row=row, qb=qb, den=den, acc=acc, nqb=nqb):
                @pl.when(cur >= 0)
                def _():
                    r = 1.0 / (den + 1e-16)
                    for i in range(DB):
                        obuf[pl.ds(row * D + i * LANES, LANES)] = acc[i] * r

                @pl.when(nqb != qb)
                def _():
                    qstart = pl.multiple_of((n0 + nqb * QB) * D, QB * D)
                    pltpu.sync_copy(q_hbm.at[pl.ds(qstart, QB * D)], qblk)

            cur = jnp.where(sw, d, cur)
            row = jnp.where(sw, d - n0, row)
            qb = jnp.where(sw, nqb, qb)
            den = jnp.where(sw, zero16, den)
            acc = [jnp.where(sw, zero16, a) for a in acc]

            qoff = jnp.clip((row - qb * QB) * D, 0, (QB - 1) * D)
            part = (qblk[pl.ds(qoff, LANES)] * kbuf[b][j, pl.ds(0, LANES)])
            for i in range(1, DB):
                part = part + (qblk[pl.ds(qoff + i * LANES, LANES)]
                               * kbuf[b][j, pl.ds(i * LANES, LANES)])
            l = _lane_bfly_sum(part) * (1.0 / math.sqrt(D))
            bsp = bb[pl.ds(row * LANES, LANES)]
            ex = jnp.exp(l - bsp)
            ex = jnp.where(valid, ex, zero16)
            den = den + ex
            for i in range(DB):
                acc[i] = acc[i] + ex * vbuf[b][j, pl.ds(i * LANES, LANES)]

        return (cur, row, qb, den, tuple(acc))

    def pair_body(pr, st):
        for bb_ in range(2):
            c = 2 * pr + bb_
            ob = 1 - bb_

            @pl.when(c + 1 < nch)
            def _():
                _ids_wait(ob)
                _gather_start(ob)

            @pl.when(c < nch)
            def _():
                _gather_wait(bb_)

            st = compute_chunk(c, bb_, st)

            @pl.when(c + 2 < nch)
            def _():
                _ids_start(c + 2, bb_)
        return st

    npairs = lax.div(nch + 1, jnp.int32(2))
    init = (jnp.int32(-1), jnp.int32(0), jnp.int32(-1), zero16,
            tuple([zero16] * DB))
    cur, row, qb, den, acc = lax.fori_loop(0, npairs, pair_body, init)

    @pl.when(cur >= 0)
    def _():
        r = 1.0 / (den + 1e-16)
        for i in range(DB):
            obuf[pl.ds(row * D + i * LANES, LANES)] = acc[i] * r

    pltpu.sync_copy(obuf, out_hbm.at[pl.ds(pl.multiple_of(n0 * D, NPW * D),
                                           NPW * D)])


def _edge_stage(q1d, k2d, v2d, srcp, dstp, rs, bnd):
    mesh = plsc.VectorSubcoreMesh(core_axis_name="c", subcore_axis_name="s")
    f = pl.kernel(
        _edge_body,
        out_type=jax.ShapeDtypeStruct((NPAD * D,), jnp.float32),
        mesh=mesh,
        scratch_types=(
            [pltpu.VMEM((QB * D,), jnp.float32),
             pltpu.VMEM((NPW * D,), jnp.float32)]
            + [pltpu.VMEM((CHK, D), jnp.float32),
               pltpu.VMEM((CHK, D), jnp.float32),
               pltpu.VMEM((CHK,), jnp.int32),
               pltpu.VMEM((CHK,), jnp.int32)] * 2
            + [pltpu.VMEM((NPW + 8,), jnp.int32),
               pltpu.VMEM((NPW * LANES,), jnp.float32)]
            + [pltpu.SemaphoreType.DMA] * 8
        ),
    )
    return f(q1d, k2d, v2d, srcp, dstp, rs, bnd)


def kernel(x, params, edge_index, batch):
    p = params
    src, dst = edge_index[0], edge_index[1]

    perm = jnp.argsort(dst)
    srcp = jnp.pad(src[perm], (0, EPAD - E))
    dstp_real = dst[perm]
    dstp = jnp.pad(dstp_real, (0, EPAD - E))
    rs = jnp.searchsorted(dstp_real, jnp.arange(NPAD + 8, dtype=jnp.int32),
                          side="left").astype(jnp.int32)
    batch_pad = jnp.pad(batch, (0, NPAD - N),
                        constant_values=G).reshape(NPAD, 1)

    xp = jnp.pad(x, ((0, NPAD - N), (0, 0)))

    wlin = jnp.concatenate(
        [p["W_lin"]] + [jnp.zeros((F, F), jnp.float32)] * 3, axis=1)
    blin = jnp.concatenate(
        [p["b_lin"]] + [jnp.zeros((F,), jnp.float32)] * 3).reshape(1, 4 * F)
    h0 = _qkvs_matmul(xp, wlin, blin)[0][:, :F]
    zeros_f = jnp.zeros((NPAD, F), jnp.float32)
    h = _bn_relu(h0, zeros_f, p["bn0_g"].reshape(1, F),
                 p["bn0_b"].reshape(1, F))

    for i in range(5):
        wcat = jnp.concatenate(
            [p["c%d_W%s" % (i, nm)] for nm in ("q", "k", "v", "s")], axis=1)
        bcat = jnp.concatenate(
            [p["c%d_b%s" % (i, nm)] for nm in ("q", "k", "v", "s")]
        ).reshape(1, 4 * D)
        q, k, v, s, qn2, kn2 = _qkvs_matmul(h, wcat, bcat)
        kmax2 = jnp.max(kn2)
        bnd = (jnp.sqrt(qn2.reshape(NPAD)) * jnp.sqrt(kmax2)
               * (1.0 / math.sqrt(D)))
        bnd = jnp.broadcast_to(bnd[:, None], (NPAD, LANES)).reshape(-1)
        attn = _edge_stage(q.reshape(NPAD * D), k, v, srcp, dstp, rs, bnd)
        h = _bn_relu(attn.reshape(NPAD, D), s,
                     p["bn%d_g" % (i + 1)].reshape(1, D),
                     p["bn%d_b" % (i + 1)].reshape(1, D))

    sums, cnts = _pool(h, batch_pad)
    wout = p["W_out"]
    bout = p["b_out"].reshape(1, CLS)
    return _head(sums, cnts, wout, bout)
